# baseline (speedup 1.0000x reference)
import functools

import jax
import jax.numpy as jnp
from jax import lax
from jax.experimental import pallas as pl
from jax.experimental.pallas import tpu as pltpu
from jax.experimental.pallas import tpu_sc as plsc

N = 10000
E = 320000
D = 128

NC = 2
NS = 16
NW = NC * NS
EPW = E // NW
CH = 125
NCHUNK = EPW // CH
NPHASE = 2
PCHUNK = NCHUNK // NPHASE
HROWS = 1000

_MESH = plsc.VectorSubcoreMesh(core_axis_name="c", subcore_axis_name="s")


@functools.partial(
    pl.kernel,
    out_type=jax.ShapeDtypeStruct((NC * N,), jnp.float32),
    mesh=_MESH,
    scratch_types=[
        pltpu.VMEM((PCHUNK, CH), jnp.int32),
        pltpu.VMEM((128,), jnp.float32),
        pltpu.VMEM((HROWS,), jnp.float32),
        pltpu.VMEM_SHARED((N,), jnp.float32),
    ],
)
def _sc_degree(dst_hbm, degp_hbm, didx, ones_v, stage, hist):
    c = lax.axis_index("c")
    s = lax.axis_index("s")
    wid = c * NS + s

    for i in range(8):
        ones_v[pl.ds(i * 16, 16)] = jnp.ones((16,), jnp.float32)

    @pl.when(s < N // HROWS)
    def _():
        for i in range(HROWS // 16):
            stage[pl.ds(i * 16, 16)] = jnp.zeros((16,), jnp.float32)
        stage[pl.ds(HROWS - 16, 16)] = jnp.zeros((16,), jnp.float32)
        pltpu.sync_copy(stage, hist.at[pl.ds(s * HROWS, HROWS)])

    plsc.subcore_barrier()

    for p in range(NPHASE):
        pltpu.sync_copy(dst_hbm.at[wid, p], didx)

        @pl.loop(0, PCHUNK)
        def _(j):
            pltpu.sync_copy(ones_v.at[pl.ds(0, CH)],
                            hist.at[didx.at[j]], add=True)

    plsc.subcore_barrier()

    @pl.when(s < N // HROWS)
    def _():
        pltpu.sync_copy(hist.at[pl.ds(s * HROWS, HROWS)], stage)
        pltpu.sync_copy(stage, degp_hbm.at[pl.ds(c * N + s * HROWS, HROWS)])


@functools.partial(
    pl.kernel,
    out_type=jax.ShapeDtypeStruct((NC, N, D), jnp.float32),
    mesh=_MESH,
    scratch_types=[
        pltpu.VMEM((PCHUNK, CH), jnp.int32),
        pltpu.VMEM((PCHUNK, CH), jnp.int32),
        pltpu.VMEM((CH, D), jnp.float32),
        pltpu.VMEM((CH, D), jnp.float32),
        pltpu.SemaphoreType.DMA,
        pltpu.SemaphoreType.DMA,
        pltpu.SemaphoreType.DMA,
        pltpu.SemaphoreType.DMA,
        pltpu.VMEM_SHARED((N, D), jnp.float32),
    ],
)
def _sc_aggregate(y_hbm, src_hbm, dst_hbm, zrows_hbm, ap_hbm,
                  sidx, didx, rows0, rows1, gsem0, gsem1, ssem0, ssem1, acc):
    c = lax.axis_index("c")
    s = lax.axis_index("s")
    wid = c * NS + s

    @pl.when(s < N // HROWS)
    def _():
        pltpu.sync_copy(zrows_hbm, rows0)
        for k in range(HROWS // CH):
            pltpu.sync_copy(rows0, acc.at[pl.ds(s * HROWS + k * CH, CH)])

    plsc.subcore_barrier()

    def g_start(j, buf, sem):
        pltpu.async_copy(y_hbm.at[sidx.at[j]], buf, sem)

    def g_wait(j, buf, sem):
        pltpu.make_async_copy(y_hbm.at[sidx.at[j]], buf, sem).wait()

    def s_start(j, buf, sem):
        pltpu.async_copy(buf, acc.at[didx.at[j]], sem, add=True)

    def s_wait(j, buf, sem):
        pltpu.make_async_copy(buf, acc.at[didx.at[j]], sem).wait()

    for p in range(NPHASE):
        pltpu.sync_copy(src_hbm.at[wid, p], sidx)
        pltpu.sync_copy(dst_hbm.at[wid, p], didx)

        g_start(0, rows0, gsem0)
        g_start(1, rows1, gsem1)

        @pl.loop(0, PCHUNK // 2 - 1)
        def _(t):
            j = 2 * t
            g_wait(j, rows0, gsem0)
            s_start(j, rows0, ssem0)
            g_wait(j + 1, rows1, gsem1)
            s_start(j + 1, rows1, ssem1)
            s_wait(j, rows0, ssem0)
            g_start(j + 2, rows0, gsem0)
            s_wait(j + 1, rows1, ssem1)
            g_start(j + 3, rows1, gsem1)

        g_wait(PCHUNK - 2, rows0, gsem0)
        s_start(PCHUNK - 2, rows0, ssem0)
        g_wait(PCHUNK - 1, rows1, gsem1)
        s_start(PCHUNK - 1, rows1, ssem1)
        s_wait(PCHUNK - 2, rows0, ssem0)
        s_wait(PCHUNK - 1, rows1, ssem1)

    plsc.subcore_barrier()

    @pl.when(s < N // HROWS)
    def _():
        pltpu.sync_copy(acc.at[pl.ds(s * HROWS, HROWS)],
                        ap_hbm.at[c, pl.ds(s * HROWS, HROWS)])


_RB = 2000


def _dis_col(dseg):
    deg = dseg[0] + dseg[1]
    return jnp.where(deg > 0.0, lax.rsqrt(jnp.maximum(deg, 1.0)), 0.0)


def _tc_matmul_body(x_ref, w_ref, degp_ref, y_ref):
    dis = _dis_col(degp_ref[...])
    xw = lax.dot_general(x_ref[...], w_ref[...], (((1,), (1,)), ((), ())),
                         preferred_element_type=jnp.float32)
    y_ref[...] = xw * dis


def _tc_matmul(x, w, degp):
    return pl.pallas_call(
        _tc_matmul_body,
        out_shape=jax.ShapeDtypeStruct((N, D), jnp.float32),
        grid=(N // _RB,),
        in_specs=[
            pl.BlockSpec((_RB, D), lambda i: (i, 0)),
            pl.BlockSpec((D, D), lambda i: (0, 0)),
            pl.BlockSpec((NC, _RB, 1), lambda i: (0, i, 0)),
        ],
        out_specs=pl.BlockSpec((_RB, D), lambda i: (i, 0)),
    )(x, w, degp)


def _tc_finalize_body(ap_ref, degp_ref, b_ref, o_ref):
    dis = _dis_col(degp_ref[...])
    agg = (ap_ref[0] + ap_ref[1]) * dis + b_ref[...]
    o_ref[...] = jnp.maximum(agg, 0.0)


def _tc_finalize(aparts, degp, b2):
    return pl.pallas_call(
        _tc_finalize_body,
        out_shape=jax.ShapeDtypeStruct((N, D), jnp.float32),
        grid=(N // _RB,),
        in_specs=[
            pl.BlockSpec((NC, _RB, D), lambda i: (0, i, 0)),
            pl.BlockSpec((NC, _RB, 1), lambda i: (0, i, 0)),
            pl.BlockSpec((1, D), lambda i: (0, 0)),
        ],
        out_specs=pl.BlockSpec((_RB, D), lambda i: (i, 0)),
    )(aparts, degp, b2)


def kernel(x, edge_index, W, b):
    src4 = edge_index[0].reshape(NW, NPHASE, PCHUNK, CH)
    dst4 = edge_index[1].reshape(NW, NPHASE, PCHUNK, CH)
    zrows = jnp.zeros((CH, D), jnp.float32)

    degp = _sc_degree(dst4).reshape(NC, N, 1)
    y = _tc_matmul(x, W, degp)
    aparts = _sc_aggregate(y, src4, dst4, zrows)
    out = _tc_finalize(aparts, degp, b.reshape(1, D))
    return out


# speedup vs baseline: 1.1951x; 1.1951x over previous
import functools

import jax
import jax.numpy as jnp
from jax import lax
from jax.experimental import pallas as pl
from jax.experimental.pallas import tpu as pltpu
from jax.experimental.pallas import tpu_sc as plsc

N = 10000
E = 320000
D = 128

NC = 2
NS = 16
NW = NC * NS
EPW = E // NW
CH = 125
NCHUNK = EPW // CH
NPHASE = 2
PCHUNK = NCHUNK // NPHASE
HROWS = 1000

_MESH = plsc.VectorSubcoreMesh(core_axis_name="c", subcore_axis_name="s")


@functools.partial(
    pl.kernel,
    out_type=jax.ShapeDtypeStruct((NC * N,), jnp.float32),
    mesh=_MESH,
    scratch_types=[
        pltpu.VMEM((PCHUNK, CH), jnp.int32),
        pltpu.VMEM((128,), jnp.float32),
        pltpu.VMEM((HROWS,), jnp.float32),
        pltpu.VMEM_SHARED((N,), jnp.float32),
    ],
)
def _sc_degree(dst_hbm, degp_hbm, didx, ones_v, stage, hist):
    c = lax.axis_index("c")
    s = lax.axis_index("s")
    wid = c * NS + s

    for i in range(8):
        ones_v[pl.ds(i * 16, 16)] = jnp.ones((16,), jnp.float32)

    @pl.when(s < N // HROWS)
    def _():
        for i in range(HROWS // 16):
            stage[pl.ds(i * 16, 16)] = jnp.zeros((16,), jnp.float32)
        stage[pl.ds(HROWS - 16, 16)] = jnp.zeros((16,), jnp.float32)
        pltpu.sync_copy(stage, hist.at[pl.ds(s * HROWS, HROWS)])

    plsc.subcore_barrier()

    for p in range(NPHASE):
        pltpu.sync_copy(dst_hbm.at[wid, p], didx)

        @pl.loop(0, PCHUNK)
        def _(j):
            pltpu.sync_copy(ones_v.at[pl.ds(0, CH)],
                            hist.at[didx.at[j]], add=True)

    plsc.subcore_barrier()

    @pl.when(s < N // HROWS)
    def _():
        pltpu.sync_copy(hist.at[pl.ds(s * HROWS, HROWS)], stage)
        pltpu.sync_copy(stage, degp_hbm.at[pl.ds(c * N + s * HROWS, HROWS)])


@functools.partial(
    pl.kernel,
    out_type=jax.ShapeDtypeStruct((NC, N, D), jnp.float32),
    mesh=_MESH,
    scratch_types=[
        pltpu.VMEM((PCHUNK, CH), jnp.int32),
        pltpu.VMEM((PCHUNK, CH), jnp.int32),
        pltpu.VMEM((CH, D), jnp.float32),
        pltpu.VMEM((CH, D), jnp.float32),
        pltpu.SemaphoreType.DMA,
        pltpu.SemaphoreType.DMA,
        pltpu.VMEM_SHARED((N, D), jnp.float32),
    ],
)
def _sc_aggregate(y_hbm, src_hbm, dst_hbm, zrows_hbm, ap_hbm,
                  sidx, didx, rows0, rows1, sem0, sem1, acc):
    c = lax.axis_index("c")
    s = lax.axis_index("s")
    wid = c * NS + s

    @pl.when(s < N // HROWS)
    def _():
        pltpu.sync_copy(zrows_hbm, rows0)
        for k in range(HROWS // CH):
            pltpu.sync_copy(rows0, acc.at[pl.ds(s * HROWS + k * CH, CH)])

    plsc.subcore_barrier()

    def g_start(j, buf, sem):
        pltpu.async_copy(y_hbm.at[sidx.at[j]], buf, sem)

    def g_wait(j, buf, sem):
        pltpu.make_async_copy(y_hbm.at[sidx.at[j]], buf, sem).wait()

    def s_add(j, buf):
        pltpu.sync_copy(buf, acc.at[didx.at[j]], add=True)

    for p in range(NPHASE):
        pltpu.sync_copy(src_hbm.at[wid, p], sidx)
        pltpu.sync_copy(dst_hbm.at[wid, p], didx)

        g_start(0, rows0, sem0)

        @pl.loop(0, PCHUNK // 2 - 1)
        def _(t):
            j = 2 * t
            g_start(j + 1, rows1, sem1)
            g_wait(j, rows0, sem0)
            s_add(j, rows0)
            g_start(j + 2, rows0, sem0)
            g_wait(j + 1, rows1, sem1)
            s_add(j + 1, rows1)

        g_start(PCHUNK - 1, rows1, sem1)
        g_wait(PCHUNK - 2, rows0, sem0)
        s_add(PCHUNK - 2, rows0)
        g_wait(PCHUNK - 1, rows1, sem1)
        s_add(PCHUNK - 1, rows1)

    plsc.subcore_barrier()

    @pl.when(s < N // HROWS)
    def _():
        pltpu.sync_copy(acc.at[pl.ds(s * HROWS, HROWS)],
                        ap_hbm.at[c, pl.ds(s * HROWS, HROWS)])


_RB = 2000


def _dis_col(dseg):
    deg = dseg[0] + dseg[1]
    return jnp.where(deg > 0.0, lax.rsqrt(jnp.maximum(deg, 1.0)), 0.0)


def _tc_matmul_body(x_ref, w_ref, y_ref):
    y_ref[...] = lax.dot_general(x_ref[...], w_ref[...],
                                 (((1,), (1,)), ((), ())),
                                 preferred_element_type=jnp.float32)


def _tc_matmul(x, w):
    return pl.pallas_call(
        _tc_matmul_body,
        out_shape=jax.ShapeDtypeStruct((N, D), jnp.float32),
        grid=(N // _RB,),
        in_specs=[
            pl.BlockSpec((_RB, D), lambda i: (i, 0)),
            pl.BlockSpec((D, D), lambda i: (0, 0)),
        ],
        out_specs=pl.BlockSpec((_RB, D), lambda i: (i, 0)),
    )(x, w)


def _tc_scale_body(xw_ref, degp_ref, y_ref):
    y_ref[...] = xw_ref[...] * _dis_col(degp_ref[...])


def _tc_scale(xw, degp):
    return pl.pallas_call(
        _tc_scale_body,
        out_shape=jax.ShapeDtypeStruct((N, D), jnp.float32),
        grid=(N // _RB,),
        in_specs=[
            pl.BlockSpec((_RB, D), lambda i: (i, 0)),
            pl.BlockSpec((NC, _RB, 1), lambda i: (0, i, 0)),
        ],
        out_specs=pl.BlockSpec((_RB, D), lambda i: (i, 0)),
    )(xw, degp)


def _tc_finalize_body(ap_ref, degp_ref, b_ref, o_ref):
    dis = _dis_col(degp_ref[...])
    agg = (ap_ref[0] + ap_ref[1]) * dis + b_ref[...]
    o_ref[...] = jnp.maximum(agg, 0.0)


def _tc_finalize(aparts, degp, b2):
    return pl.pallas_call(
        _tc_finalize_body,
        out_shape=jax.ShapeDtypeStruct((N, D), jnp.float32),
        grid=(N // _RB,),
        in_specs=[
            pl.BlockSpec((NC, _RB, D), lambda i: (0, i, 0)),
            pl.BlockSpec((NC, _RB, 1), lambda i: (0, i, 0)),
            pl.BlockSpec((1, D), lambda i: (0, 0)),
        ],
        out_specs=pl.BlockSpec((_RB, D), lambda i: (i, 0)),
    )(aparts, degp, b2)


def kernel(x, edge_index, W, b):
    src4 = edge_index[0].reshape(NW, NPHASE, PCHUNK, CH)
    dst4 = edge_index[1].reshape(NW, NPHASE, PCHUNK, CH)
    zrows = jnp.zeros((CH, D), jnp.float32)

    degp = _sc_degree(dst4).reshape(NC, N, 1)
    xw = _tc_matmul(x, W)
    y = _tc_scale(xw, degp)
    aparts = _sc_aggregate(y, src4, dst4, zrows)
    out = _tc_finalize(aparts, degp, b.reshape(1, D))
    return out


# speedup vs baseline: 1.2055x; 1.0087x over previous
import functools

import jax
import jax.numpy as jnp
from jax import lax
from jax.experimental import pallas as pl
from jax.experimental.pallas import tpu as pltpu
from jax.experimental.pallas import tpu_sc as plsc

N = 10000
E = 320000
D = 128

NC = 2
NS = 16
NW = NC * NS
EPW = E // NW
CH = 125
NCHUNK = EPW // CH
NPHASE = 2
PCHUNK = NCHUNK // NPHASE
HROWS = 1000

_MESH = plsc.VectorSubcoreMesh(core_axis_name="c", subcore_axis_name="s")


@functools.partial(
    pl.kernel,
    out_type=jax.ShapeDtypeStruct((NC * N,), jnp.float32),
    mesh=_MESH,
    scratch_types=[
        pltpu.VMEM((PCHUNK, CH), jnp.int32),
        pltpu.VMEM((128,), jnp.float32),
        pltpu.VMEM((HROWS,), jnp.float32),
        pltpu.VMEM_SHARED((N,), jnp.float32),
    ],
)
def _sc_degree(dst_hbm, degp_hbm, didx, ones_v, stage, hist):
    c = lax.axis_index("c")
    s = lax.axis_index("s")
    wid = c * NS + s

    for i in range(8):
        ones_v[pl.ds(i * 16, 16)] = jnp.ones((16,), jnp.float32)

    @pl.when(s < N // HROWS)
    def _():
        for i in range(HROWS // 16):
            stage[pl.ds(i * 16, 16)] = jnp.zeros((16,), jnp.float32)
        stage[pl.ds(HROWS - 16, 16)] = jnp.zeros((16,), jnp.float32)
        pltpu.sync_copy(stage, hist.at[pl.ds(s * HROWS, HROWS)])

    plsc.subcore_barrier()

    for p in range(NPHASE):
        pltpu.sync_copy(dst_hbm.at[wid, p], didx)

        @pl.loop(0, PCHUNK)
        def _(j):
            pltpu.sync_copy(ones_v.at[pl.ds(0, CH)],
                            hist.at[didx.at[j]], add=True)

    plsc.subcore_barrier()

    @pl.when(s < N // HROWS)
    def _():
        pltpu.sync_copy(hist.at[pl.ds(s * HROWS, HROWS)], stage)
        pltpu.sync_copy(stage, degp_hbm.at[pl.ds(c * N + s * HROWS, HROWS)])


@functools.partial(
    pl.kernel,
    out_type=jax.ShapeDtypeStruct((NC, N, D), jnp.float32),
    mesh=_MESH,
    scratch_types=[
        pltpu.VMEM((PCHUNK, CH), jnp.int32),
        pltpu.VMEM((PCHUNK, CH), jnp.int32),
        pltpu.VMEM((CH, D), jnp.float32),
        pltpu.VMEM((CH, D), jnp.float32),
        pltpu.SemaphoreType.DMA,
        pltpu.SemaphoreType.DMA,
        pltpu.VMEM_SHARED((N, D), jnp.float32),
    ],
)
def _sc_aggregate(y_hbm, src_hbm, dst_hbm, zrows_hbm, ap_hbm,
                  sidx, didx, rows0, rows1, sem0, sem1, acc):
    c = lax.axis_index("c")
    s = lax.axis_index("s")
    wid = c * NS + s

    @pl.when(s < N // HROWS)
    def _():
        pltpu.async_copy(zrows_hbm, rows0, sem0)

    pltpu.sync_copy(src_hbm.at[wid, 0], sidx)
    pltpu.sync_copy(dst_hbm.at[wid, 0], didx)

    @pl.when(s < N // HROWS)
    def _():
        pltpu.make_async_copy(zrows_hbm, rows0, sem0).wait()
        for k in range(HROWS // CH):
            pltpu.async_copy(rows0, acc.at[pl.ds(s * HROWS + k * CH, CH)],
                             sem0)
        for k in range(HROWS // CH):
            pltpu.make_async_copy(
                rows0, acc.at[pl.ds(s * HROWS + k * CH, CH)], sem0).wait()

    plsc.subcore_barrier()

    def g_start(j, buf, sem):
        pltpu.async_copy(y_hbm.at[sidx.at[j]], buf, sem)

    def g_wait(j, buf, sem):
        pltpu.make_async_copy(y_hbm.at[sidx.at[j]], buf, sem).wait()

    def s_add(j, buf):
        pltpu.sync_copy(buf, acc.at[didx.at[j]], add=True)

    for p in range(NPHASE):
        if p > 0:
            pltpu.sync_copy(src_hbm.at[wid, p], sidx)
            pltpu.sync_copy(dst_hbm.at[wid, p], didx)

        g_start(0, rows0, sem0)

        @pl.loop(0, PCHUNK // 2 - 1)
        def _(t):
            j = 2 * t
            g_start(j + 1, rows1, sem1)
            g_wait(j, rows0, sem0)
            s_add(j, rows0)
            g_start(j + 2, rows0, sem0)
            g_wait(j + 1, rows1, sem1)
            s_add(j + 1, rows1)

        g_start(PCHUNK - 1, rows1, sem1)
        g_wait(PCHUNK - 2, rows0, sem0)
        s_add(PCHUNK - 2, rows0)
        g_wait(PCHUNK - 1, rows1, sem1)
        s_add(PCHUNK - 1, rows1)

    plsc.subcore_barrier()

    @pl.when(s < N // HROWS)
    def _():
        pltpu.sync_copy(acc.at[pl.ds(s * HROWS, HROWS)],
                        ap_hbm.at[c, pl.ds(s * HROWS, HROWS)])


_RB = 2000


def _dis_col(dseg):
    deg = dseg[0] + dseg[1]
    return jnp.where(deg > 0.0, lax.rsqrt(jnp.maximum(deg, 1.0)), 0.0)


def _tc_matmul_body(x_ref, w_ref, degp_ref, y_ref):
    dis = _dis_col(degp_ref[...])
    xw = lax.dot_general(x_ref[...], w_ref[...], (((1,), (1,)), ((), ())),
                         preferred_element_type=jnp.float32)
    y_ref[...] = xw * dis


def _tc_matmul(x, w, degp):
    return pl.pallas_call(
        _tc_matmul_body,
        out_shape=jax.ShapeDtypeStruct((N, D), jnp.float32),
        grid=(N // _RB,),
        in_specs=[
            pl.BlockSpec((_RB, D), lambda i: (i, 0)),
            pl.BlockSpec((D, D), lambda i: (0, 0)),
            pl.BlockSpec((NC, _RB, 1), lambda i: (0, i, 0)),
        ],
        out_specs=pl.BlockSpec((_RB, D), lambda i: (i, 0)),
    )(x, w, degp)


def _tc_finalize_body(ap_ref, degp_ref, b_ref, o_ref):
    dis = _dis_col(degp_ref[...])
    agg = (ap_ref[0] + ap_ref[1]) * dis + b_ref[...]
    o_ref[...] = jnp.maximum(agg, 0.0)


def _tc_finalize(aparts, degp, b2):
    return pl.pallas_call(
        _tc_finalize_body,
        out_shape=jax.ShapeDtypeStruct((N, D), jnp.float32),
        grid=(N // _RB,),
        in_specs=[
            pl.BlockSpec((NC, _RB, D), lambda i: (0, i, 0)),
            pl.BlockSpec((NC, _RB, 1), lambda i: (0, i, 0)),
            pl.BlockSpec((1, D), lambda i: (0, 0)),
        ],
        out_specs=pl.BlockSpec((_RB, D), lambda i: (i, 0)),
    )(aparts, degp, b2)


def kernel(x, edge_index, W, b):
    src4 = edge_index[0].reshape(NW, NPHASE, PCHUNK, CH)
    dst4 = edge_index[1].reshape(NW, NPHASE, PCHUNK, CH)
    zrows = jnp.zeros((CH, D), jnp.float32)

    degp = _sc_degree(dst4).reshape(NC, N, 1)
    y = _tc_matmul(x, W, degp)
    aparts = _sc_aggregate(y, src4, dst4, zrows)
    out = _tc_finalize(aparts, degp, b.reshape(1, D))
    return out


# speedup vs baseline: 1.2353x; 1.0247x over previous
import functools

import jax
import jax.numpy as jnp
from jax import lax
from jax.experimental import pallas as pl
from jax.experimental.pallas import tpu as pltpu
from jax.experimental.pallas import tpu_sc as plsc

N = 10000
E = 320000
D = 128

NC = 2
NS = 16
NW = NC * NS
EPW = E // NW
CH = 125
NCHUNK = EPW // CH
NPHASE = 2
PCHUNK = NCHUNK // NPHASE
HROWS = 1000

_MESH = plsc.VectorSubcoreMesh(core_axis_name="c", subcore_axis_name="s")


@functools.partial(
    pl.kernel,
    out_type=jax.ShapeDtypeStruct((NC * N,), jnp.float32),
    mesh=_MESH,
    scratch_types=[
        pltpu.VMEM((PCHUNK, CH), jnp.int32),
        pltpu.VMEM((128,), jnp.float32),
        pltpu.VMEM((HROWS,), jnp.float32),
        pltpu.SemaphoreType.DMA,
        pltpu.VMEM_SHARED((N,), jnp.float32),
    ],
)
def _sc_degree(dst_hbm, degp_hbm, didx, ones_v, stage, sem, hist):
    c = lax.axis_index("c")
    s = lax.axis_index("s")
    wid = c * NS + s

    for i in range(8):
        ones_v[pl.ds(i * 16, 16)] = jnp.ones((16,), jnp.float32)

    @pl.when(s < N // HROWS)
    def _():
        for i in range(HROWS // 16):
            stage[pl.ds(i * 16, 16)] = jnp.zeros((16,), jnp.float32)
        stage[pl.ds(HROWS - 16, 16)] = jnp.zeros((16,), jnp.float32)
        pltpu.sync_copy(stage, hist.at[pl.ds(s * HROWS, HROWS)])

    plsc.subcore_barrier()

    for p in range(NPHASE):
        pltpu.sync_copy(dst_hbm.at[wid, p], didx)

        @pl.loop(0, PCHUNK)
        def _(j):
            pltpu.async_copy(ones_v.at[pl.ds(0, CH)],
                             hist.at[didx.at[j]], sem, add=True)

        @pl.loop(0, PCHUNK)
        def _(j):
            pltpu.make_async_copy(ones_v.at[pl.ds(0, CH)],
                                  hist.at[didx.at[j]], sem).wait()

    plsc.subcore_barrier()

    @pl.when(s < N // HROWS)
    def _():
        pltpu.sync_copy(hist.at[pl.ds(s * HROWS, HROWS)], stage)
        pltpu.sync_copy(stage, degp_hbm.at[pl.ds(c * N + s * HROWS, HROWS)])


@functools.partial(
    pl.kernel,
    out_type=jax.ShapeDtypeStruct((NC, N, D), jnp.float32),
    mesh=_MESH,
    scratch_types=[
        pltpu.VMEM((PCHUNK, CH), jnp.int32),
        pltpu.VMEM((PCHUNK, CH), jnp.int32),
        pltpu.VMEM((CH, D), jnp.float32),
        pltpu.VMEM((CH, D), jnp.float32),
        pltpu.SemaphoreType.DMA,
        pltpu.SemaphoreType.DMA,
        pltpu.VMEM_SHARED((N, D), jnp.float32),
    ],
)
def _sc_aggregate(y_hbm, src_hbm, dst_hbm, zrows_hbm, ap_hbm,
                  sidx, didx, rows0, rows1, sem0, sem1, acc):
    c = lax.axis_index("c")
    s = lax.axis_index("s")
    wid = c * NS + s

    @pl.when(s < N // HROWS)
    def _():
        pltpu.async_copy(zrows_hbm, rows0, sem0)

    pltpu.sync_copy(src_hbm.at[wid, 0], sidx)
    pltpu.sync_copy(dst_hbm.at[wid, 0], didx)

    @pl.when(s < N // HROWS)
    def _():
        pltpu.make_async_copy(zrows_hbm, rows0, sem0).wait()
        for k in range(HROWS // CH):
            pltpu.async_copy(rows0, acc.at[pl.ds(s * HROWS + k * CH, CH)],
                             sem0)
        for k in range(HROWS // CH):
            pltpu.make_async_copy(
                rows0, acc.at[pl.ds(s * HROWS + k * CH, CH)], sem0).wait()

    plsc.subcore_barrier()

    def g_start(j, buf, sem):
        pltpu.async_copy(y_hbm.at[sidx.at[j]], buf, sem)

    def g_wait(j, buf, sem):
        pltpu.make_async_copy(y_hbm.at[sidx.at[j]], buf, sem).wait()

    def s_add(j, buf):
        pltpu.sync_copy(buf, acc.at[didx.at[j]], add=True)

    for p in range(NPHASE):
        if p > 0:
            pltpu.sync_copy(src_hbm.at[wid, p], sidx)
            pltpu.sync_copy(dst_hbm.at[wid, p], didx)

        g_start(0, rows0, sem0)

        @pl.loop(0, PCHUNK // 2 - 1)
        def _(t):
            j = 2 * t
            g_start(j + 1, rows1, sem1)
            g_wait(j, rows0, sem0)
            s_add(j, rows0)
            g_start(j + 2, rows0, sem0)
            g_wait(j + 1, rows1, sem1)
            s_add(j + 1, rows1)

        g_start(PCHUNK - 1, rows1, sem1)
        g_wait(PCHUNK - 2, rows0, sem0)
        s_add(PCHUNK - 2, rows0)
        g_wait(PCHUNK - 1, rows1, sem1)
        s_add(PCHUNK - 1, rows1)

    plsc.subcore_barrier()

    @pl.when(s < N // HROWS)
    def _():
        pltpu.sync_copy(acc.at[pl.ds(s * HROWS, HROWS)],
                        ap_hbm.at[c, pl.ds(s * HROWS, HROWS)])


_RB = 2000


def _dis_col(dseg):
    deg = dseg[0] + dseg[1]
    return jnp.where(deg > 0.0, lax.rsqrt(jnp.maximum(deg, 1.0)), 0.0)


def _tc_matmul_body(x_ref, w_ref, degp_ref, y_ref):
    dis = _dis_col(degp_ref[...])
    xw = lax.dot_general(x_ref[...], w_ref[...], (((1,), (1,)), ((), ())),
                         preferred_element_type=jnp.float32)
    y_ref[...] = xw * dis


def _tc_matmul(x, w, degp):
    return pl.pallas_call(
        _tc_matmul_body,
        out_shape=jax.ShapeDtypeStruct((N, D), jnp.float32),
        grid=(N // _RB,),
        in_specs=[
            pl.BlockSpec((_RB, D), lambda i: (i, 0)),
            pl.BlockSpec((D, D), lambda i: (0, 0)),
            pl.BlockSpec((NC, _RB, 1), lambda i: (0, i, 0)),
        ],
        out_specs=pl.BlockSpec((_RB, D), lambda i: (i, 0)),
    )(x, w, degp)


def _tc_finalize_body(ap_ref, degp_ref, b_ref, o_ref):
    dis = _dis_col(degp_ref[...])
    agg = (ap_ref[0] + ap_ref[1]) * dis + b_ref[...]
    o_ref[...] = jnp.maximum(agg, 0.0)


def _tc_finalize(aparts, degp, b2):
    return pl.pallas_call(
        _tc_finalize_body,
        out_shape=jax.ShapeDtypeStruct((N, D), jnp.float32),
        grid=(N // _RB,),
        in_specs=[
            pl.BlockSpec((NC, _RB, D), lambda i: (0, i, 0)),
            pl.BlockSpec((NC, _RB, 1), lambda i: (0, i, 0)),
            pl.BlockSpec((1, D), lambda i: (0, 0)),
        ],
        out_specs=pl.BlockSpec((_RB, D), lambda i: (i, 0)),
    )(aparts, degp, b2)


def kernel(x, edge_index, W, b):
    src4 = edge_index[0].reshape(NW, NPHASE, PCHUNK, CH)
    dst4 = edge_index[1].reshape(NW, NPHASE, PCHUNK, CH)
    zrows = jnp.zeros((CH, D), jnp.float32)

    degp = _sc_degree(dst4).reshape(NC, N, 1)
    y = _tc_matmul(x, W, degp)
    aparts = _sc_aggregate(y, src4, dst4, zrows)
    out = _tc_finalize(aparts, degp, b.reshape(1, D))
    return out


# speedup vs baseline: 1.2378x; 1.0020x over previous
import functools

import jax
import jax.numpy as jnp
from jax import lax
from jax.experimental import pallas as pl
from jax.experimental.pallas import tpu as pltpu
from jax.experimental.pallas import tpu_sc as plsc

N = 10000
E = 320000
D = 128

NC = 2
NS = 16
NW = NC * NS
EPW = E // NW
CH = 125
NCHUNK = EPW // CH
NPHASE = 2
PCHUNK = NCHUNK // NPHASE
HROWS = 1000
ZROWS = 624
ZCH = 104

_MESH = plsc.VectorSubcoreMesh(core_axis_name="c", subcore_axis_name="s")


@functools.partial(
    pl.kernel,
    out_type=jax.ShapeDtypeStruct((NC * N,), jnp.float32),
    mesh=_MESH,
    scratch_types=[
        pltpu.VMEM((PCHUNK, CH), jnp.int32),
        pltpu.VMEM((128,), jnp.float32),
        pltpu.VMEM((HROWS,), jnp.float32),
        pltpu.SemaphoreType.DMA,
        pltpu.VMEM_SHARED((N,), jnp.float32),
    ],
)
def _sc_degree(dst_hbm, degp_hbm, didx, ones_v, stage, sem, hist):
    c = lax.axis_index("c")
    s = lax.axis_index("s")
    wid = c * NS + s

    for i in range(8):
        ones_v[pl.ds(i * 16, 16)] = jnp.ones((16,), jnp.float32)

    @pl.when(s < N // HROWS)
    def _():
        for i in range(HROWS // 16):
            stage[pl.ds(i * 16, 16)] = jnp.zeros((16,), jnp.float32)
        stage[pl.ds(HROWS - 16, 16)] = jnp.zeros((16,), jnp.float32)
        pltpu.sync_copy(stage, hist.at[pl.ds(s * HROWS, HROWS)])

    plsc.subcore_barrier()

    for p in range(NPHASE):
        pltpu.sync_copy(dst_hbm.at[wid, p], didx)

        @pl.loop(0, PCHUNK)
        def _(j):
            pltpu.async_copy(ones_v.at[pl.ds(0, CH)],
                             hist.at[didx.at[j]], sem, add=True)

        @pl.loop(0, PCHUNK)
        def _(j):
            pltpu.make_async_copy(ones_v.at[pl.ds(0, CH)],
                                  hist.at[didx.at[j]], sem).wait()

    plsc.subcore_barrier()

    @pl.when(s < N // HROWS)
    def _():
        pltpu.sync_copy(hist.at[pl.ds(s * HROWS, HROWS)], stage)
        pltpu.sync_copy(stage, degp_hbm.at[pl.ds(c * N + s * HROWS, HROWS)])


@functools.partial(
    pl.kernel,
    out_type=jax.ShapeDtypeStruct((NC, N, D), jnp.float32),
    mesh=_MESH,
    scratch_types=[
        pltpu.VMEM((PCHUNK, CH), jnp.int32),
        pltpu.VMEM((PCHUNK, CH), jnp.int32),
        pltpu.VMEM((CH, D), jnp.float32),
        pltpu.VMEM((CH, D), jnp.float32),
        pltpu.SemaphoreType.DMA,
        pltpu.SemaphoreType.DMA,
        pltpu.VMEM_SHARED((N, D), jnp.float32),
    ],
)
def _sc_aggregate(y_hbm, src_hbm, dst_hbm, zrows_hbm, ap_hbm,
                  sidx, didx, rows0, rows1, sem0, sem1, acc):
    c = lax.axis_index("c")
    s = lax.axis_index("s")
    wid = c * NS + s

    pltpu.async_copy(zrows_hbm, rows0, sem0)
    pltpu.sync_copy(src_hbm.at[wid, 0], sidx)
    pltpu.sync_copy(dst_hbm.at[wid, 0], didx)

    pltpu.make_async_copy(zrows_hbm, rows0, sem0).wait()
    zoff = s * ZROWS
    for k in range(ZROWS // ZCH):
        pltpu.async_copy(rows0.at[pl.ds(0, ZCH)],
                         acc.at[pl.ds(zoff + k * ZCH, ZCH)], sem0)
    for k in range(ZROWS // ZCH):
        pltpu.make_async_copy(rows0.at[pl.ds(0, ZCH)],
                              acc.at[pl.ds(zoff + k * ZCH, ZCH)], sem0).wait()

    @pl.when(s == NS - 1)
    def _():
        pltpu.sync_copy(rows0.at[pl.ds(0, N - NS * ZROWS)],
                        acc.at[pl.ds(NS * ZROWS, N - NS * ZROWS)])

    plsc.subcore_barrier()

    def g_start(j, buf, sem):
        pltpu.async_copy(y_hbm.at[sidx.at[j]], buf, sem)

    def g_wait(j, buf, sem):
        pltpu.make_async_copy(y_hbm.at[sidx.at[j]], buf, sem).wait()

    def s_add(j, buf):
        pltpu.sync_copy(buf, acc.at[didx.at[j]], add=True)

    for p in range(NPHASE):
        if p > 0:
            pltpu.sync_copy(src_hbm.at[wid, p], sidx)
            pltpu.sync_copy(dst_hbm.at[wid, p], didx)

        g_start(0, rows0, sem0)

        @pl.loop(0, PCHUNK // 2 - 1)
        def _(t):
            j = 2 * t
            g_start(j + 1, rows1, sem1)
            g_wait(j, rows0, sem0)
            s_add(j, rows0)
            g_start(j + 2, rows0, sem0)
            g_wait(j + 1, rows1, sem1)
            s_add(j + 1, rows1)

        g_start(PCHUNK - 1, rows1, sem1)
        g_wait(PCHUNK - 2, rows0, sem0)
        s_add(PCHUNK - 2, rows0)
        g_wait(PCHUNK - 1, rows1, sem1)
        s_add(PCHUNK - 1, rows1)

    plsc.subcore_barrier()

    @pl.when(s < NS - 1)
    def _():
        pltpu.sync_copy(acc.at[pl.ds(s * ZROWS, ZROWS)],
                        ap_hbm.at[c, pl.ds(s * ZROWS, ZROWS)])

    @pl.when(s == NS - 1)
    def _():
        pltpu.sync_copy(acc.at[pl.ds((NS - 1) * ZROWS, N - (NS - 1) * ZROWS)],
                        ap_hbm.at[c, pl.ds((NS - 1) * ZROWS,
                                           N - (NS - 1) * ZROWS)])


_RB = 2000


def _dis_col(dseg):
    deg = dseg[0] + dseg[1]
    return jnp.where(deg > 0.0, lax.rsqrt(jnp.maximum(deg, 1.0)), 0.0)


def _tc_matmul_body(x_ref, w_ref, degp_ref, y_ref):
    dis = _dis_col(degp_ref[...])
    xw = lax.dot_general(x_ref[...], w_ref[...], (((1,), (1,)), ((), ())),
                         preferred_element_type=jnp.float32)
    y_ref[...] = xw * dis


def _tc_matmul(x, w, degp):
    return pl.pallas_call(
        _tc_matmul_body,
        out_shape=jax.ShapeDtypeStruct((N, D), jnp.float32),
        grid=(N // _RB,),
        in_specs=[
            pl.BlockSpec((_RB, D), lambda i: (i, 0)),
            pl.BlockSpec((D, D), lambda i: (0, 0)),
            pl.BlockSpec((NC, _RB, 1), lambda i: (0, i, 0)),
        ],
        out_specs=pl.BlockSpec((_RB, D), lambda i: (i, 0)),
    )(x, w, degp)


def _tc_finalize_body(ap_ref, degp_ref, b_ref, o_ref):
    dis = _dis_col(degp_ref[...])
    agg = (ap_ref[0] + ap_ref[1]) * dis + b_ref[...]
    o_ref[...] = jnp.maximum(agg, 0.0)


def _tc_finalize(aparts, degp, b2):
    return pl.pallas_call(
        _tc_finalize_body,
        out_shape=jax.ShapeDtypeStruct((N, D), jnp.float32),
        grid=(N // _RB,),
        in_specs=[
            pl.BlockSpec((NC, _RB, D), lambda i: (0, i, 0)),
            pl.BlockSpec((NC, _RB, 1), lambda i: (0, i, 0)),
            pl.BlockSpec((1, D), lambda i: (0, 0)),
        ],
        out_specs=pl.BlockSpec((_RB, D), lambda i: (i, 0)),
    )(aparts, degp, b2)


def kernel(x, edge_index, W, b):
    src4 = edge_index[0].reshape(NW, NPHASE, PCHUNK, CH)
    dst4 = edge_index[1].reshape(NW, NPHASE, PCHUNK, CH)
    zrows = jnp.zeros((CH, D), jnp.float32)

    degp = _sc_degree(dst4).reshape(NC, N, 1)
    y = _tc_matmul(x, W, degp)
    aparts = _sc_aggregate(y, src4, dst4, zrows)
    out = _tc_finalize(aparts, degp, b.reshape(1, D))
    return out
